# baseline (speedup 1.0000x reference)
import functools
import math

import jax
import jax.numpy as jnp
from jax import lax
from jax.experimental import pallas as pl
from jax.experimental.pallas import tpu as pltpu
from jax.experimental.pallas import tpu_sc as plsc

N = 50000
E = 800000
F = 128
D = 16
H = 64
B = 64
G = 36
L = 4
NG = 20
NT = 9

BN = 512
NBLK = math.ceil(N / BN)
N_PAD = NBLK * BN
BE = 512
EBLK = math.ceil(E / (16 * BE)) * 16
E_PAD = EBLK * BE
NEG = -3.0e38


def _ln(v, s, b):
    m = jnp.mean(v, axis=-1, keepdims=True)
    var = jnp.mean((v - m) ** 2, axis=-1, keepdims=True)
    return (v - m) * jax.lax.rsqrt(var + 1e-5) * s + b


def _embed_body(x_ref, w_ref, b_ref, s_ref, lb_ref, wx_ref, h_ref,
                hx0_ref, hx1_ref):
    h = jnp.maximum(x_ref[...] @ w_ref[...] + b_ref[...], 0.0)
    h = _ln(h, s_ref[...], lb_ref[...])
    h_ref[...] = h
    hh = h @ wx_ref[...]
    hx0_ref[...] = hh[:, :32]
    hx1_ref[...] = hh[:, 32:]


def _embed(x, w, b, s, lb, wx):
    return pl.pallas_call(
        _embed_body,
        grid=(NBLK,),
        in_specs=[
            pl.BlockSpec((BN, F), lambda i: (i, 0)),
            pl.BlockSpec((F, H), lambda i: (0, 0)),
            pl.BlockSpec((1, H), lambda i: (0, 0)),
            pl.BlockSpec((1, H), lambda i: (0, 0)),
            pl.BlockSpec((1, H), lambda i: (0, 0)),
            pl.BlockSpec((H, H), lambda i: (0, 0)),
        ],
        out_specs=[
            pl.BlockSpec((BN, H), lambda i: (i, 0)),
            pl.BlockSpec((BN, 32), lambda i: (i, 0)),
            pl.BlockSpec((BN, 32), lambda i: (i, 0)),
        ],
        out_shape=[
            jax.ShapeDtypeStruct((N, H), jnp.float32),
            jax.ShapeDtypeStruct((N, 32), jnp.float32),
            jax.ShapeDtypeStruct((N, 32), jnp.float32),
        ],
    )(x, w, b, s, lb, wx)


BE2 = 1000


def _ea_body(ea_ref, egt_ref, wa_ref, gt_ref, o0, o1, o2, o3):
    bi = lax.broadcasted_iota(jnp.int32, (BE2, NG), 1)
    onehot = (egt_ref[...] == bi).astype(jnp.float32)
    res = ea_ref[...] @ wa_ref[...] + onehot @ gt_ref[...]
    for l, o in enumerate((o0, o1, o2, o3)):
        o[0] = res[:, 64 * l:64 * l + 32]
        o[1] = res[:, 64 * l + 32:64 * l + 64]


def _ea_proj(ea_p, egt_p, wa_cat, gt_cat):
    return pl.pallas_call(
        _ea_body,
        grid=(E // BE2,),
        in_specs=[
            pl.BlockSpec((BE2, D), lambda i: (i, 0)),
            pl.BlockSpec((BE2, 1), lambda i: (i, 0)),
            pl.BlockSpec((D, 4 * H), lambda i: (0, 0)),
            pl.BlockSpec((NG, 4 * H), lambda i: (0, 0)),
        ],
        out_specs=[pl.BlockSpec((2, BE2, 32), lambda i: (0, i, 0))] * 4,
        out_shape=[jax.ShapeDtypeStruct((2, E_PAD, 32), jnp.float32)] * 4,
    )(ea_p, egt_p, wa_cat, gt_cat)


def _upd_body(h_ref, a0_ref, a1_ref, d0_ref, d1_ref, u1h_ref, w2u_ref,
              bvec_ref, u2_ref, b2u2_ref, lns_ref, lnb_ref, wx_ref,
              h_out, hx0_out, hx1_out):
    h = h_ref[...]
    w2u = w2u_ref[...]
    bv = bvec_ref[...]
    deg = d0_ref[0][:, :1] + d1_ref[0][:, :1]
    t = (h @ u1h_ref[...] + a0_ref[0] @ w2u[:32] + a1_ref[0] @ w2u[32:]
         + deg * bv[0:1] + bv[1:2])
    t = jnp.maximum(t, 0.0)
    u = t @ u2_ref[...] + b2u2_ref[...]
    u = _ln(u, lns_ref[...], lnb_ref[...])
    hn = h + u
    h_out[...] = hn
    hh = hn @ wx_ref[...]
    hx0_out[...] = hh[:, :32]
    hx1_out[...] = hh[:, 32:]


def _update(h, aggp, deg, u1h, w2u, bvec, u2, b2u2, lns, lnb, wx):
    return pl.pallas_call(
        _upd_body,
        grid=(NBLK,),
        in_specs=[
            pl.BlockSpec((BN, H), lambda i: (i, 0)),
            pl.BlockSpec((1, BN, 32), lambda i: (0, i, 0)),
            pl.BlockSpec((1, BN, 32), lambda i: (1, i, 0)),
            pl.BlockSpec((1, BN, 8), lambda i: (0, i, 0)),
            pl.BlockSpec((1, BN, 8), lambda i: (1, i, 0)),
            pl.BlockSpec((H, H), lambda i: (0, 0)),
            pl.BlockSpec((H, H), lambda i: (0, 0)),
            pl.BlockSpec((2, H), lambda i: (0, 0)),
            pl.BlockSpec((H, H), lambda i: (0, 0)),
            pl.BlockSpec((1, H), lambda i: (0, 0)),
            pl.BlockSpec((1, H), lambda i: (0, 0)),
            pl.BlockSpec((1, H), lambda i: (0, 0)),
            pl.BlockSpec((H, H), lambda i: (0, 0)),
        ],
        out_specs=[
            pl.BlockSpec((BN, H), lambda i: (i, 0)),
            pl.BlockSpec((BN, 32), lambda i: (i, 0)),
            pl.BlockSpec((BN, 32), lambda i: (i, 0)),
        ],
        out_shape=[
            jax.ShapeDtypeStruct((N, H), jnp.float32),
            jax.ShapeDtypeStruct((N, 32), jnp.float32),
            jax.ShapeDtypeStruct((N, 32), jnp.float32),
        ],
    )(h, aggp, aggp, deg, deg, u1h, w2u, bvec, u2, b2u2, lns, lnb, wx)


def _updpool_body(h_ref, a0_ref, a1_ref, d0_ref, d1_ref, u1h_ref, w2u_ref,
                  bvec_ref, u2_ref, b2u2_ref, lns_ref, lnb_ref, batch_ref,
                  s_out, m_out, c_out):
    i = pl.program_id(0)
    h = h_ref[...]
    w2u = w2u_ref[...]
    bv = bvec_ref[...]
    deg = d0_ref[0][:, :1] + d1_ref[0][:, :1]
    t = (h @ u1h_ref[...] + a0_ref[0] @ w2u[:32] + a1_ref[0] @ w2u[32:]
         + deg * bv[0:1] + bv[1:2])
    t = jnp.maximum(t, 0.0)
    u = t @ u2_ref[...] + b2u2_ref[...]
    u = _ln(u, lns_ref[...], lnb_ref[...])
    hn = h + u

    @pl.when(i == 0)
    def _():
        s_out[...] = jnp.zeros_like(s_out)
        m_out[...] = jnp.full_like(m_out, NEG)
        c_out[...] = jnp.zeros_like(c_out)

    rows = i * BN + lax.broadcasted_iota(jnp.int32, (BN, 1), 0)
    valid = rows < N
    hn = jnp.where(valid, hn, 0.0)
    bids = batch_ref[...]
    bi = lax.broadcasted_iota(jnp.int32, (BN, B), 1)
    msk = ((bids == bi) & valid).astype(jnp.float32)
    s_out[...] += lax.dot_general(msk, hn, (((0,), (0,)), ((), ())))
    c_out[...] += lax.dot_general(msk, jnp.ones((BN, H), jnp.float32),
                                  (((0,), (0,)), ((), ())))
    bmin = bids[0, 0]
    bmax = bids[BN - 1, 0]

    def body(b, _):
        sel = jnp.where((bids == b) & valid, hn, NEG)
        mx = jnp.max(sel, axis=0, keepdims=True)
        m_out[pl.ds(b, 1), :] = jnp.maximum(m_out[pl.ds(b, 1), :], mx)
        return 0

    lax.fori_loop(bmin, bmax + 1, body, 0)


def _update_pool(h, aggp, deg, u1h, w2u, bvec, u2, b2u2, lns, lnb, batch_p):
    return pl.pallas_call(
        _updpool_body,
        grid=(NBLK,),
        in_specs=[
            pl.BlockSpec((BN, H), lambda i: (i, 0)),
            pl.BlockSpec((1, BN, 32), lambda i: (0, i, 0)),
            pl.BlockSpec((1, BN, 32), lambda i: (1, i, 0)),
            pl.BlockSpec((1, BN, 8), lambda i: (0, i, 0)),
            pl.BlockSpec((1, BN, 8), lambda i: (1, i, 0)),
            pl.BlockSpec((H, H), lambda i: (0, 0)),
            pl.BlockSpec((H, H), lambda i: (0, 0)),
            pl.BlockSpec((2, H), lambda i: (0, 0)),
            pl.BlockSpec((H, H), lambda i: (0, 0)),
            pl.BlockSpec((1, H), lambda i: (0, 0)),
            pl.BlockSpec((1, H), lambda i: (0, 0)),
            pl.BlockSpec((1, H), lambda i: (0, 0)),
            pl.BlockSpec((BN, 1), lambda i: (i, 0)),
        ],
        out_specs=[
            pl.BlockSpec((B, H), lambda i: (0, 0)),
            pl.BlockSpec((B, H), lambda i: (0, 0)),
            pl.BlockSpec((B, H), lambda i: (0, 0)),
        ],
        out_shape=[
            jax.ShapeDtypeStruct((B, H), jnp.float32),
            jax.ShapeDtypeStruct((B, H), jnp.float32),
            jax.ShapeDtypeStruct((B, H), jnp.float32),
        ],
    )(h, aggp, aggp, deg, deg, u1h, w2u, bvec, u2, b2u2, lns, lnb, batch_p)


def _head_body(s_ref, m_ref, c_ref, gf_ref, gw_ref, gb_ref, gs_ref, glb_ref,
               w1_ref, b1_ref, w2_ref, b2_ref, wh_ref, bh_ref, out_ref):
    cnt = c_ref[...]
    h_sum = s_ref[...]
    mean = h_sum / jnp.maximum(cnt, 1.0)
    hmax = jnp.where(cnt > 0, m_ref[...], 0.0)
    g = jnp.maximum(gf_ref[...] @ gw_ref[...] + gb_ref[...], 0.0)
    g = _ln(g, gs_ref[...], glb_ref[...])
    w1 = w1_ref[...]
    c1 = (mean @ w1[:H] + hmax @ w1[H:2 * H] + h_sum @ w1[2 * H:3 * H]
          + g @ w1[3 * H:] + b1_ref[...])
    c1 = jnp.maximum(c1, 0.0)
    c2 = jnp.maximum(c1 @ w2_ref[...] + b2_ref[...], 0.0)
    out_ref[...] = c2 @ wh_ref[...] + bh_ref[...]


def _head(h_sum, h_max, cnt, gf, gw, gb, gs, glb, w1, b1, w2, b2, wh, bh):
    return pl.pallas_call(
        _head_body,
        out_shape=jax.ShapeDtypeStruct((B, 128), jnp.float32),
    )(h_sum, h_max, cnt, gf, gw, gb, gs, glb, w1, b1, w2, b2, wh, bh)


BS = 256
_SC_EC = E_PAD // 16
_SC_CH = _SC_EC // BS
_ROWS_SUB = N_PAD // 16
_MESH = plsc.VectorSubcoreMesh(core_axis_name="c", subcore_axis_name="s")


@functools.partial(
    pl.kernel,
    out_type=jax.ShapeDtypeStruct((2, N_PAD, 32), jnp.float32),
    mesh=_MESH,
    compiler_params=pltpu.CompilerParams(use_tc_tiling_on_sc=False),
    scratch_types=[
        pltpu.VMEM((2, 2, 128), jnp.int32),
        pltpu.VMEM((2, 2, 128), jnp.int32),
        pltpu.VMEM((2, BS, 32), jnp.float32),
        pltpu.VMEM((BS, 32), jnp.float32),
        pltpu.VMEM_SHARED((N_PAD, 32), jnp.float32),
        pltpu.SemaphoreType.DMA,
        pltpu.SemaphoreType.DMA,
    ],
)
def _edge_sc(hx0_hbm, hx1_hbm, ea_hbm, src_hbm, dst_hbm, zer_hbm, out_hbm,
             sidx, didx, rows, eab, acc, gsem, ssem):
    c = lax.axis_index("c")
    s = lax.axis_index("s")

    pltpu.sync_copy(zer_hbm, rows.at[0])
    base = s * _ROWS_SUB
    for t in range(_ROWS_SUB // BS):
        pltpu.sync_copy(rows.at[0], acc.at[pl.ds(base + BS * t, BS)])
    rem = _ROWS_SUB % BS
    if rem:
        pltpu.sync_copy(rows.at[0, pl.ds(0, rem)],
                        acc.at[pl.ds(base + (_ROWS_SUB // BS) * BS, rem)])
    plsc.subcore_barrier()

    def load_idx(g, b):
        ebase = s * _SC_EC + g * BS
        for j in range(2):
            pltpu.sync_copy(src_hbm.at[pl.ds(ebase + j * 128, 128)],
                            sidx.at[b, j])
            pltpu.sync_copy(dst_hbm.at[pl.ds(ebase + j * 128, 128)],
                            didx.at[b, j])

    def issue_gather(b):
        @pl.when(c == 0)
        def _():
            for j in range(2):
                pltpu.async_copy(hx0_hbm.at[sidx.at[b, j]],
                                 rows.at[b, pl.ds(j * 128, 128)], gsem)

        @pl.when(c == 1)
        def _():
            for j in range(2):
                pltpu.async_copy(hx1_hbm.at[sidx.at[b, j]],
                                 rows.at[b, pl.ds(j * 128, 128)], gsem)

    load_idx(0, 0)
    issue_gather(0)
    pltpu.sync_copy(ea_hbm.at[c, pl.ds(s * _SC_EC, BS)], eab)

    def pair(k, carry):
        for b in (0, 1):
            g = 2 * k + b
            nb = 1 - b
            pltpu.make_async_copy(zer_hbm, rows.at[b], gsem).wait()

            @pl.when(g + 1 < _SC_CH)
            def _():
                load_idx(g + 1, nb)
                issue_gather(nb)

            @plsc.parallel_loop(0, BS, unroll=8)
            def _relu(r):
                for hh in range(2):
                    sl = pl.ds(hh * 16, 16)
                    rows[b, r, sl] = jnp.maximum(rows[b, r, sl] + eab[r, sl],
                                                 0.0)

            scps = [pltpu.async_copy(rows.at[b, pl.ds(j * 128, 128)],
                                     acc.at[didx.at[b, j]], ssem, add=True)
                    for j in range(2)]

            @pl.when(g + 1 < _SC_CH)
            def _():
                pltpu.sync_copy(
                    ea_hbm.at[c, pl.ds(s * _SC_EC + (g + 1) * BS, BS)], eab)

            for cp in scps:
                cp.wait()
        return carry

    lax.fori_loop(0, _SC_CH // 2, pair, 0)
    plsc.subcore_barrier()

    for t in range(_ROWS_SUB // BS):
        pltpu.sync_copy(acc.at[pl.ds(base + BS * t, BS)], rows.at[0])
        pltpu.sync_copy(rows.at[0], out_hbm.at[c, pl.ds(base + BS * t, BS)])
    if rem:
        pltpu.sync_copy(acc.at[pl.ds(base + (_ROWS_SUB // BS) * BS, rem)],
                        rows.at[0, pl.ds(0, rem)])
        pltpu.sync_copy(rows.at[0, pl.ds(0, rem)],
                        out_hbm.at[c, pl.ds(base + (_ROWS_SUB // BS) * BS, rem)])


_DG_EW = E_PAD // 32
_DG_CH = _DG_EW // BS


@functools.partial(
    pl.kernel,
    out_type=jax.ShapeDtypeStruct((2, N_PAD, 8), jnp.float32),
    mesh=_MESH,
    compiler_params=pltpu.CompilerParams(use_tc_tiling_on_sc=False),
    scratch_types=[
        pltpu.VMEM((2, 128), jnp.int32),
        pltpu.VMEM((128, 8), jnp.float32),
        pltpu.VMEM((BS, 8), jnp.float32),
        pltpu.VMEM_SHARED((N_PAD, 8), jnp.float32),
        pltpu.SemaphoreType.DMA,
    ],
)
def _deg_sc(dst_hbm, ones_hbm, zer_hbm, out_hbm, didx, ones, stage, acc, sem):
    c = lax.axis_index("c")
    s = lax.axis_index("s")
    wid = s * 2 + c
    pltpu.sync_copy(ones_hbm, ones)
    pltpu.sync_copy(zer_hbm, stage)
    base = s * _ROWS_SUB
    for t in range(_ROWS_SUB // BS):
        pltpu.sync_copy(stage, acc.at[pl.ds(base + BS * t, BS)])
    rem = _ROWS_SUB % BS
    if rem:
        pltpu.sync_copy(stage.at[pl.ds(0, rem)],
                        acc.at[pl.ds(base + (_ROWS_SUB // BS) * BS, rem)])
    plsc.subcore_barrier()

    def chunk(g, carry):
        ebase = wid * _DG_EW + g * BS
        for j in range(2):
            pltpu.sync_copy(dst_hbm.at[pl.ds(ebase + j * 128, 128)],
                            didx.at[j])
        for j in range(2):
            pltpu.sync_copy(ones, acc.at[didx.at[j]], add=True)
        return carry

    lax.fori_loop(0, _DG_CH, chunk, 0)
    plsc.subcore_barrier()

    for t in range(_ROWS_SUB // BS):
        pltpu.sync_copy(acc.at[pl.ds(base + BS * t, BS)], stage)
        pltpu.sync_copy(stage, out_hbm.at[c, pl.ds(base + BS * t, BS)])
    if rem:
        pltpu.sync_copy(acc.at[pl.ds(base + (_ROWS_SUB // BS) * BS, rem)],
                        stage.at[pl.ds(0, rem)])
        pltpu.sync_copy(stage.at[pl.ds(0, rem)],
                        out_hbm.at[c, pl.ds(base + (_ROWS_SUB // BS) * BS, rem)])


def kernel(x, edge_index, edge_attr, edge_gate_type, batch, global_features, params):
    p = params
    src = edge_index[0]
    dst = edge_index[1]
    src_p = jnp.pad(src, (0, E_PAD - E))
    dst_p = jnp.pad(dst, (0, E_PAD - E), constant_values=N)
    egt_p = edge_gate_type[:, None]
    ea_p = edge_attr
    batch_p = jnp.pad(batch, (0, N_PAD - N), constant_values=B - 1)[:, None]

    w1 = p['msg_W1']
    w1x = w1[:, :H]
    wa_cat = jnp.concatenate([w1[l, 2 * H:] for l in range(L)], axis=1)
    gt_cat = jnp.concatenate(
        [p['gate_embed'][l] @ w1[l, H:2 * H] + p['msg_b1'][l][None, :]
         for l in range(L)], axis=1)
    u1 = p['upd_W1']
    u1h = u1[:, :H]
    w2u = jnp.einsum('lij,ljk->lik', p['msg_W2'], u1[:, H:])
    b2u = jnp.einsum('lj,ljk->lk', p['msg_b2'], u1[:, H:])
    bvec = jnp.stack([b2u, p['upd_b1']], axis=1)
    wh = jnp.zeros((H, 128), jnp.float32)
    wh = wh.at[:, :NT].set(p['thr_W']).at[:, NT:NT + 1].set(p['run_W'])
    bh = jnp.zeros((1, 128), jnp.float32)
    bh = bh.at[0, :NT].set(p['thr_b']).at[0, NT:NT + 1].set(p['run_b'])

    def r2(v):
        return v[None, :]

    h, hx0, hx1 = _embed(x, p['embed_W'], r2(p['embed_b']),
                         r2(p['embed_ln_s']), r2(p['embed_ln_b']), w1x[0])
    ea_layers = _ea_proj(ea_p, egt_p, wa_cat, gt_cat)
    deg8 = _deg_sc(dst_p, jnp.ones((128, 8), jnp.float32),
                   jnp.zeros((BS, 8), jnp.float32))
    zer32 = jnp.zeros((BS, 32), jnp.float32)

    for l in range(L):
        aggp = _edge_sc(hx0, hx1, ea_layers[l], src_p, dst_p, zer32)
        args = (h, aggp, deg8, u1h[l], w2u[l], bvec[l], p['upd_W2'][l],
                r2(p['upd_b2'][l]), r2(p['mp_ln_s'][l]), r2(p['mp_ln_b'][l]))
        if l < L - 1:
            h, hx0, hx1 = _update(*args, w1x[l + 1])
        else:
            h_sum, h_max, cnt = _update_pool(*args, batch_p)

    out = _head(h_sum, h_max, cnt, global_features, p['glob_W'],
                r2(p['glob_b']), r2(p['glob_ln_s']), r2(p['glob_ln_b']),
                p['comb_W1'], jnp.zeros((1, 2 * H), jnp.float32) + p['comb_b1'],
                p['comb_W2'], r2(p['comb_b2']), wh, bh)
    return out[:, :NT], out[:, NT]


# speedup vs baseline: 1.2914x; 1.2914x over previous
import functools
import math

import jax
import jax.numpy as jnp
from jax import lax
from jax.experimental import pallas as pl
from jax.experimental.pallas import tpu as pltpu
from jax.experimental.pallas import tpu_sc as plsc

N = 50000
E = 800000
F = 128
D = 16
H = 64
B = 64
G = 36
L = 4
NG = 20
NT = 9

BN = 512
NBLK = math.ceil(N / BN)
N_PAD = NBLK * BN
BE = 512
EBLK = math.ceil(E / (16 * BE)) * 16
E_PAD = EBLK * BE
NEG = -3.0e38


def _ln(v, s, b):
    m = jnp.mean(v, axis=-1, keepdims=True)
    var = jnp.mean((v - m) ** 2, axis=-1, keepdims=True)
    return (v - m) * jax.lax.rsqrt(var + 1e-5) * s + b


def _embed_body(x_ref, w_ref, b_ref, s_ref, lb_ref, wx_ref, h_ref,
                hx0_ref, hx1_ref):
    h = jnp.maximum(x_ref[...] @ w_ref[...] + b_ref[...], 0.0)
    h = _ln(h, s_ref[...], lb_ref[...])
    h_ref[...] = h
    hh = h @ wx_ref[...]
    hx0_ref[...] = hh[:, :32]
    hx1_ref[...] = hh[:, 32:]


def _embed(x, w, b, s, lb, wx):
    return pl.pallas_call(
        _embed_body,
        grid=(NBLK,),
        in_specs=[
            pl.BlockSpec((BN, F), lambda i: (i, 0)),
            pl.BlockSpec((F, H), lambda i: (0, 0)),
            pl.BlockSpec((1, H), lambda i: (0, 0)),
            pl.BlockSpec((1, H), lambda i: (0, 0)),
            pl.BlockSpec((1, H), lambda i: (0, 0)),
            pl.BlockSpec((H, H), lambda i: (0, 0)),
        ],
        out_specs=[
            pl.BlockSpec((BN, H), lambda i: (i, 0)),
            pl.BlockSpec((BN, 32), lambda i: (i, 0)),
            pl.BlockSpec((BN, 32), lambda i: (i, 0)),
        ],
        out_shape=[
            jax.ShapeDtypeStruct((N, H), jnp.float32),
            jax.ShapeDtypeStruct((N, 32), jnp.float32),
            jax.ShapeDtypeStruct((N, 32), jnp.float32),
        ],
    )(x, w, b, s, lb, wx)


BE2 = 800
E4 = E_PAD // 4


def _ea_body(ea4_ref, wbd_ref, gtbd_ref, o0, o1, o2, o3):
    blk = ea4_ref[...]
    gi = lax.broadcasted_iota(jnp.int32, (BE2 // 4, NG), 1).astype(jnp.float32)
    oh = jnp.concatenate(
        [(blk[:, 64 + ss:65 + ss] == gi).astype(jnp.float32)
         for ss in range(4)], axis=1)
    res = blk[:, :64] @ wbd_ref[...] + oh @ gtbd_ref[...]
    for l, o in enumerate((o0, o1, o2, o3)):
        o[0] = res[:, (2 * l) * 128:(2 * l) * 128 + 128]
        o[1] = res[:, (2 * l + 1) * 128:(2 * l + 1) * 128 + 128]


def _ea_proj(eaE, wbd, gtbd):
    return pl.pallas_call(
        _ea_body,
        grid=(E // BE2,),
        in_specs=[
            pl.BlockSpec((BE2 // 4, 68), lambda i: (i, 0)),
            pl.BlockSpec((64, 8 * 128), lambda i: (0, 0)),
            pl.BlockSpec((4 * NG, 8 * 128), lambda i: (0, 0)),
        ],
        out_specs=[pl.BlockSpec((2, BE2 // 4, 128), lambda i: (0, i, 0))] * 4,
        out_shape=[jax.ShapeDtypeStruct((2, E4, 128), jnp.float32)] * 4,
    )(eaE, wbd, gtbd)


def _upd_body(h_ref, a0_ref, a1_ref, d0_ref, d1_ref, u1h_ref, w2u_ref,
              bvec_ref, u2_ref, b2u2_ref, lns_ref, lnb_ref, wx_ref,
              h_out, hx0_out, hx1_out):
    h = h_ref[...]
    w2u = w2u_ref[...]
    bv = bvec_ref[...]
    deg = d0_ref[0][:, :1] + d1_ref[0][:, :1]
    t = (h @ u1h_ref[...] + a0_ref[0] @ w2u[:32] + a1_ref[0] @ w2u[32:]
         + deg * bv[0:1] + bv[1:2])
    t = jnp.maximum(t, 0.0)
    u = t @ u2_ref[...] + b2u2_ref[...]
    u = _ln(u, lns_ref[...], lnb_ref[...])
    hn = h + u
    h_out[...] = hn
    hh = hn @ wx_ref[...]
    hx0_out[...] = hh[:, :32]
    hx1_out[...] = hh[:, 32:]


def _update(h, aggp, deg, u1h, w2u, bvec, u2, b2u2, lns, lnb, wx):
    return pl.pallas_call(
        _upd_body,
        grid=(NBLK,),
        in_specs=[
            pl.BlockSpec((BN, H), lambda i: (i, 0)),
            pl.BlockSpec((1, BN, 32), lambda i: (0, i, 0)),
            pl.BlockSpec((1, BN, 32), lambda i: (1, i, 0)),
            pl.BlockSpec((1, BN, 8), lambda i: (0, i, 0)),
            pl.BlockSpec((1, BN, 8), lambda i: (1, i, 0)),
            pl.BlockSpec((H, H), lambda i: (0, 0)),
            pl.BlockSpec((H, H), lambda i: (0, 0)),
            pl.BlockSpec((2, H), lambda i: (0, 0)),
            pl.BlockSpec((H, H), lambda i: (0, 0)),
            pl.BlockSpec((1, H), lambda i: (0, 0)),
            pl.BlockSpec((1, H), lambda i: (0, 0)),
            pl.BlockSpec((1, H), lambda i: (0, 0)),
            pl.BlockSpec((H, H), lambda i: (0, 0)),
        ],
        out_specs=[
            pl.BlockSpec((BN, H), lambda i: (i, 0)),
            pl.BlockSpec((BN, 32), lambda i: (i, 0)),
            pl.BlockSpec((BN, 32), lambda i: (i, 0)),
        ],
        out_shape=[
            jax.ShapeDtypeStruct((N, H), jnp.float32),
            jax.ShapeDtypeStruct((N, 32), jnp.float32),
            jax.ShapeDtypeStruct((N, 32), jnp.float32),
        ],
    )(h, aggp, aggp, deg, deg, u1h, w2u, bvec, u2, b2u2, lns, lnb, wx)


def _updpool_body(h_ref, a0_ref, a1_ref, d0_ref, d1_ref, u1h_ref, w2u_ref,
                  bvec_ref, u2_ref, b2u2_ref, lns_ref, lnb_ref, batch_ref,
                  s_out, m_out, c_out):
    i = pl.program_id(0)
    h = h_ref[...]
    w2u = w2u_ref[...]
    bv = bvec_ref[...]
    deg = d0_ref[0][:, :1] + d1_ref[0][:, :1]
    t = (h @ u1h_ref[...] + a0_ref[0] @ w2u[:32] + a1_ref[0] @ w2u[32:]
         + deg * bv[0:1] + bv[1:2])
    t = jnp.maximum(t, 0.0)
    u = t @ u2_ref[...] + b2u2_ref[...]
    u = _ln(u, lns_ref[...], lnb_ref[...])
    hn = h + u

    @pl.when(i == 0)
    def _():
        s_out[...] = jnp.zeros_like(s_out)
        m_out[...] = jnp.full_like(m_out, NEG)
        c_out[...] = jnp.zeros_like(c_out)

    rows = i * BN + lax.broadcasted_iota(jnp.int32, (BN, 1), 0)
    valid = rows < N
    hn = jnp.where(valid, hn, 0.0)
    bids = batch_ref[...]
    bi = lax.broadcasted_iota(jnp.int32, (BN, B), 1)
    msk = ((bids == bi) & valid).astype(jnp.float32)
    s_out[...] += lax.dot_general(msk, hn, (((0,), (0,)), ((), ())))
    c_out[...] += lax.dot_general(msk, jnp.ones((BN, H), jnp.float32),
                                  (((0,), (0,)), ((), ())))
    bmin = bids[0, 0]
    bmax = bids[BN - 1, 0]

    def body(b, _):
        sel = jnp.where((bids == b) & valid, hn, NEG)
        mx = jnp.max(sel, axis=0, keepdims=True)
        m_out[pl.ds(b, 1), :] = jnp.maximum(m_out[pl.ds(b, 1), :], mx)
        return 0

    lax.fori_loop(bmin, bmax + 1, body, 0)


def _update_pool(h, aggp, deg, u1h, w2u, bvec, u2, b2u2, lns, lnb, batch_p):
    return pl.pallas_call(
        _updpool_body,
        grid=(NBLK,),
        in_specs=[
            pl.BlockSpec((BN, H), lambda i: (i, 0)),
            pl.BlockSpec((1, BN, 32), lambda i: (0, i, 0)),
            pl.BlockSpec((1, BN, 32), lambda i: (1, i, 0)),
            pl.BlockSpec((1, BN, 8), lambda i: (0, i, 0)),
            pl.BlockSpec((1, BN, 8), lambda i: (1, i, 0)),
            pl.BlockSpec((H, H), lambda i: (0, 0)),
            pl.BlockSpec((H, H), lambda i: (0, 0)),
            pl.BlockSpec((2, H), lambda i: (0, 0)),
            pl.BlockSpec((H, H), lambda i: (0, 0)),
            pl.BlockSpec((1, H), lambda i: (0, 0)),
            pl.BlockSpec((1, H), lambda i: (0, 0)),
            pl.BlockSpec((1, H), lambda i: (0, 0)),
            pl.BlockSpec((BN, 1), lambda i: (i, 0)),
        ],
        out_specs=[
            pl.BlockSpec((B, H), lambda i: (0, 0)),
            pl.BlockSpec((B, H), lambda i: (0, 0)),
            pl.BlockSpec((B, H), lambda i: (0, 0)),
        ],
        out_shape=[
            jax.ShapeDtypeStruct((B, H), jnp.float32),
            jax.ShapeDtypeStruct((B, H), jnp.float32),
            jax.ShapeDtypeStruct((B, H), jnp.float32),
        ],
    )(h, aggp, aggp, deg, deg, u1h, w2u, bvec, u2, b2u2, lns, lnb, batch_p)


def _head_body(s_ref, m_ref, c_ref, gf_ref, gw_ref, gb_ref, gs_ref, glb_ref,
               w1_ref, b1_ref, w2_ref, b2_ref, wh_ref, bh_ref, out_ref):
    cnt = c_ref[...]
    h_sum = s_ref[...]
    mean = h_sum / jnp.maximum(cnt, 1.0)
    hmax = jnp.where(cnt > 0, m_ref[...], 0.0)
    g = jnp.maximum(gf_ref[...] @ gw_ref[...] + gb_ref[...], 0.0)
    g = _ln(g, gs_ref[...], glb_ref[...])
    w1 = w1_ref[...]
    c1 = (mean @ w1[:H] + hmax @ w1[H:2 * H] + h_sum @ w1[2 * H:3 * H]
          + g @ w1[3 * H:] + b1_ref[...])
    c1 = jnp.maximum(c1, 0.0)
    c2 = jnp.maximum(c1 @ w2_ref[...] + b2_ref[...], 0.0)
    out_ref[...] = c2 @ wh_ref[...] + bh_ref[...]


def _head(h_sum, h_max, cnt, gf, gw, gb, gs, glb, w1, b1, w2, b2, wh, bh):
    return pl.pallas_call(
        _head_body,
        out_shape=jax.ShapeDtypeStruct((B, 128), jnp.float32),
    )(h_sum, h_max, cnt, gf, gw, gb, gs, glb, w1, b1, w2, b2, wh, bh)


BS = 256
_SC_EC = E_PAD // 16
_SC_CH = _SC_EC // BS
_ROWS_SUB = N_PAD // 16
_MESH = plsc.VectorSubcoreMesh(core_axis_name="c", subcore_axis_name="s")


@functools.partial(
    pl.kernel,
    out_type=jax.ShapeDtypeStruct((2, N_PAD, 32), jnp.float32),
    mesh=_MESH,
    compiler_params=pltpu.CompilerParams(use_tc_tiling_on_sc=False),
    scratch_types=[
        pltpu.VMEM((2, 2, 128), jnp.int32),
        pltpu.VMEM((2, 2, 128), jnp.int32),
        pltpu.VMEM((2, BS, 32), jnp.float32),
        pltpu.VMEM((BS // 4, 128), jnp.float32),
        pltpu.VMEM_SHARED((N_PAD, 32), jnp.float32),
        pltpu.SemaphoreType.DMA,
        pltpu.SemaphoreType.DMA,
    ],
)
def _edge_sc(hx0_hbm, hx1_hbm, ea_hbm, src_hbm, dst_hbm, zer_hbm, out_hbm,
             sidx, didx, rows, eab, acc, gsem, ssem):
    c = lax.axis_index("c")
    s = lax.axis_index("s")

    pltpu.sync_copy(zer_hbm, rows.at[0])
    base = s * _ROWS_SUB
    for t in range(_ROWS_SUB // BS):
        pltpu.sync_copy(rows.at[0], acc.at[pl.ds(base + BS * t, BS)])
    rem = _ROWS_SUB % BS
    if rem:
        pltpu.sync_copy(rows.at[0, pl.ds(0, rem)],
                        acc.at[pl.ds(base + (_ROWS_SUB // BS) * BS, rem)])
    plsc.subcore_barrier()

    def load_idx(g, b):
        ebase = s * _SC_EC + g * BS
        for j in range(2):
            pltpu.sync_copy(src_hbm.at[pl.ds(ebase + j * 128, 128)],
                            sidx.at[b, j])
            pltpu.sync_copy(dst_hbm.at[pl.ds(ebase + j * 128, 128)],
                            didx.at[b, j])

    def issue_gather(b):
        @pl.when(c == 0)
        def _():
            for j in range(2):
                pltpu.async_copy(hx0_hbm.at[sidx.at[b, j]],
                                 rows.at[b, pl.ds(j * 128, 128)], gsem)

        @pl.when(c == 1)
        def _():
            for j in range(2):
                pltpu.async_copy(hx1_hbm.at[sidx.at[b, j]],
                                 rows.at[b, pl.ds(j * 128, 128)], gsem)

    load_idx(0, 0)
    issue_gather(0)
    pltpu.sync_copy(ea_hbm.at[c, pl.ds(s * _SC_EC // 4, BS // 4)], eab)

    def pair(k, carry):
        for b in (0, 1):
            g = 2 * k + b
            nb = 1 - b
            pltpu.make_async_copy(zer_hbm, rows.at[b], gsem).wait()

            @pl.when(g + 1 < _SC_CH)
            def _():
                load_idx(g + 1, nb)
                issue_gather(nb)

            @plsc.parallel_loop(0, BS, unroll=8)
            def _relu(r):
                for hh in range(2):
                    p = r * 32 + hh * 16
                    sl = pl.ds(hh * 16, 16)
                    rows[b, r, sl] = jnp.maximum(
                        rows[b, r, sl] + eab[p // 128, pl.ds(p % 128, 16)],
                        0.0)

            scps = [pltpu.async_copy(rows.at[b, pl.ds(j * 128, 128)],
                                     acc.at[didx.at[b, j]], ssem, add=True)
                    for j in range(2)]

            @pl.when(g + 1 < _SC_CH)
            def _():
                pltpu.sync_copy(
                    ea_hbm.at[c, pl.ds((s * _SC_EC + (g + 1) * BS) // 4,
                                       BS // 4)], eab)

            for cp in scps:
                cp.wait()
        return carry

    lax.fori_loop(0, _SC_CH // 2, pair, 0)
    plsc.subcore_barrier()

    for t in range(_ROWS_SUB // BS):
        pltpu.sync_copy(acc.at[pl.ds(base + BS * t, BS)], rows.at[0])
        pltpu.sync_copy(rows.at[0], out_hbm.at[c, pl.ds(base + BS * t, BS)])
    if rem:
        pltpu.sync_copy(acc.at[pl.ds(base + (_ROWS_SUB // BS) * BS, rem)],
                        rows.at[0, pl.ds(0, rem)])
        pltpu.sync_copy(rows.at[0, pl.ds(0, rem)],
                        out_hbm.at[c, pl.ds(base + (_ROWS_SUB // BS) * BS, rem)])


_DG_EW = E_PAD // 32
_DG_CH = _DG_EW // BS


@functools.partial(
    pl.kernel,
    out_type=jax.ShapeDtypeStruct((2, N_PAD, 8), jnp.float32),
    mesh=_MESH,
    compiler_params=pltpu.CompilerParams(use_tc_tiling_on_sc=False),
    scratch_types=[
        pltpu.VMEM((2, 128), jnp.int32),
        pltpu.VMEM((128, 8), jnp.float32),
        pltpu.VMEM((BS, 8), jnp.float32),
        pltpu.VMEM_SHARED((N_PAD, 8), jnp.float32),
        pltpu.SemaphoreType.DMA,
    ],
)
def _deg_sc(dst_hbm, ones_hbm, zer_hbm, out_hbm, didx, ones, stage, acc, sem):
    c = lax.axis_index("c")
    s = lax.axis_index("s")
    wid = s * 2 + c
    pltpu.sync_copy(ones_hbm, ones)
    pltpu.sync_copy(zer_hbm, stage)
    base = s * _ROWS_SUB
    for t in range(_ROWS_SUB // BS):
        pltpu.sync_copy(stage, acc.at[pl.ds(base + BS * t, BS)])
    rem = _ROWS_SUB % BS
    if rem:
        pltpu.sync_copy(stage.at[pl.ds(0, rem)],
                        acc.at[pl.ds(base + (_ROWS_SUB // BS) * BS, rem)])
    plsc.subcore_barrier()

    def chunk(g, carry):
        ebase = wid * _DG_EW + g * BS
        for j in range(2):
            pltpu.sync_copy(dst_hbm.at[pl.ds(ebase + j * 128, 128)],
                            didx.at[j])
        for j in range(2):
            pltpu.sync_copy(ones, acc.at[didx.at[j]], add=True)
        return carry

    lax.fori_loop(0, _DG_CH, chunk, 0)
    plsc.subcore_barrier()

    for t in range(_ROWS_SUB // BS):
        pltpu.sync_copy(acc.at[pl.ds(base + BS * t, BS)], stage)
        pltpu.sync_copy(stage, out_hbm.at[c, pl.ds(base + BS * t, BS)])
    if rem:
        pltpu.sync_copy(acc.at[pl.ds(base + (_ROWS_SUB // BS) * BS, rem)],
                        stage.at[pl.ds(0, rem)])
        pltpu.sync_copy(stage.at[pl.ds(0, rem)],
                        out_hbm.at[c, pl.ds(base + (_ROWS_SUB // BS) * BS, rem)])


def kernel(x, edge_index, edge_attr, edge_gate_type, batch, global_features, params):
    p = params
    src = edge_index[0]
    dst = edge_index[1]
    src_p = jnp.pad(src, (0, E_PAD - E))
    dst_p = jnp.pad(dst, (0, E_PAD - E), constant_values=N)
    ea4 = edge_attr.reshape(E // 4, 4 * D)
    egt4f = edge_gate_type.astype(jnp.float32).reshape(E // 4, 4)
    eaE = jnp.concatenate([ea4, egt4f], axis=1)
    batch_p = jnp.pad(batch, (0, N_PAD - N), constant_values=B - 1)[:, None]

    w1 = p['msg_W1']
    w1x = w1[:, :H]
    wbd = jnp.zeros((4 * D, 8, 4, 32), jnp.float32)
    for l in range(L):
        for cc in range(2):
            for ss in range(4):
                wbd = wbd.at[D * ss:D * (ss + 1), 2 * l + cc, ss].set(
                    w1[l, 2 * H:, 32 * cc:32 * (cc + 1)])
    wbd = wbd.reshape(4 * D, 8 * 128)
    gtbd = jnp.zeros((4 * NG, 8, 4, 32), jnp.float32)
    for l in range(L):
        gtab = p['gate_embed'][l] @ w1[l, H:2 * H] + p['msg_b1'][l][None, :]
        for cc in range(2):
            for ss in range(4):
                gtbd = gtbd.at[NG * ss:NG * (ss + 1), 2 * l + cc, ss].set(
                    gtab[:, 32 * cc:32 * (cc + 1)])
    gtbd = gtbd.reshape(4 * NG, 8 * 128)
    u1 = p['upd_W1']
    u1h = u1[:, :H]
    w2u = jnp.einsum('lij,ljk->lik', p['msg_W2'], u1[:, H:])
    b2u = jnp.einsum('lj,ljk->lk', p['msg_b2'], u1[:, H:])
    bvec = jnp.stack([b2u, p['upd_b1']], axis=1)
    wh = jnp.zeros((H, 128), jnp.float32)
    wh = wh.at[:, :NT].set(p['thr_W']).at[:, NT:NT + 1].set(p['run_W'])
    bh = jnp.zeros((1, 128), jnp.float32)
    bh = bh.at[0, :NT].set(p['thr_b']).at[0, NT:NT + 1].set(p['run_b'])

    def r2(v):
        return v[None, :]

    h, hx0, hx1 = _embed(x, p['embed_W'], r2(p['embed_b']),
                         r2(p['embed_ln_s']), r2(p['embed_ln_b']), w1x[0])
    ea_layers = _ea_proj(eaE, wbd, gtbd)
    deg8 = _deg_sc(dst_p, jnp.ones((128, 8), jnp.float32),
                   jnp.zeros((BS, 8), jnp.float32))
    zer32 = jnp.zeros((BS, 32), jnp.float32)

    for l in range(L):
        aggp = _edge_sc(hx0, hx1, ea_layers[l], src_p, dst_p, zer32)
        args = (h, aggp, deg8, u1h[l], w2u[l], bvec[l], p['upd_W2'][l],
                r2(p['upd_b2'][l]), r2(p['mp_ln_s'][l]), r2(p['mp_ln_b'][l]))
        if l < L - 1:
            h, hx0, hx1 = _update(*args, w1x[l + 1])
        else:
            h_sum, h_max, cnt = _update_pool(*args, batch_p)

    out = _head(h_sum, h_max, cnt, global_features, p['glob_W'],
                r2(p['glob_b']), r2(p['glob_ln_s']), r2(p['glob_ln_b']),
                p['comb_W1'], jnp.zeros((1, 2 * H), jnp.float32) + p['comb_b1'],
                p['comb_W2'], r2(p['comb_b2']), wh, bh)
    return out[:, :NT], out[:, NT]


# speedup vs baseline: 1.6703x; 1.2934x over previous
import functools
import math

import jax
import jax.numpy as jnp
from jax import lax
from jax.experimental import pallas as pl
from jax.experimental.pallas import tpu as pltpu
from jax.experimental.pallas import tpu_sc as plsc

N = 50000
E = 800000
F = 128
D = 16
H = 64
B = 64
G = 36
L = 4
NG = 20
NT = 9

BN = 512
NBLK = math.ceil(N / BN)
N_PAD = NBLK * BN
BE = 512
EBLK = math.ceil(E / (16 * BE)) * 16
E_PAD = EBLK * BE
NEG = -3.0e38


def _ln(v, s, b):
    m = jnp.mean(v, axis=-1, keepdims=True)
    var = jnp.mean((v - m) ** 2, axis=-1, keepdims=True)
    return (v - m) * jax.lax.rsqrt(var + 1e-5) * s + b


def _embed_body(x_ref, w_ref, b_ref, s_ref, lb_ref, wx_ref, h_ref,
                hx0_ref, hx1_ref):
    h = jnp.maximum(x_ref[...] @ w_ref[...] + b_ref[...], 0.0)
    h = _ln(h, s_ref[...], lb_ref[...])
    h_ref[...] = h
    hh = h @ wx_ref[...]
    hx0_ref[...] = hh[:, :32]
    hx1_ref[...] = hh[:, 32:]


def _embed(x, w, b, s, lb, wx):
    return pl.pallas_call(
        _embed_body,
        grid=(NBLK,),
        in_specs=[
            pl.BlockSpec((BN, F), lambda i: (i, 0)),
            pl.BlockSpec((F, H), lambda i: (0, 0)),
            pl.BlockSpec((1, H), lambda i: (0, 0)),
            pl.BlockSpec((1, H), lambda i: (0, 0)),
            pl.BlockSpec((1, H), lambda i: (0, 0)),
            pl.BlockSpec((H, H), lambda i: (0, 0)),
        ],
        out_specs=[
            pl.BlockSpec((BN, H), lambda i: (i, 0)),
            pl.BlockSpec((BN, 32), lambda i: (i, 0)),
            pl.BlockSpec((BN, 32), lambda i: (i, 0)),
        ],
        out_shape=[
            jax.ShapeDtypeStruct((N, H), jnp.float32),
            jax.ShapeDtypeStruct((N, 32), jnp.float32),
            jax.ShapeDtypeStruct((N, 32), jnp.float32),
        ],
    )(x, w, b, s, lb, wx)


BE2 = 800
E4 = E_PAD // 4


def _ea_body(ea4_ref, wbd_ref, gtbd_ref, o0, o1, o2, o3):
    blk = ea4_ref[...]
    gi = lax.broadcasted_iota(jnp.int32, (BE2 // 4, NG), 1).astype(jnp.float32)
    oh = jnp.concatenate(
        [(blk[:, 64 + ss:65 + ss] == gi).astype(jnp.float32)
         for ss in range(4)], axis=1)
    res = blk[:, :64] @ wbd_ref[...] + oh @ gtbd_ref[...]
    for l, o in enumerate((o0, o1, o2, o3)):
        o[0] = res[:, (2 * l) * 128:(2 * l) * 128 + 128]
        o[1] = res[:, (2 * l + 1) * 128:(2 * l + 1) * 128 + 128]


def _ea_proj(eaE, wbd, gtbd):
    return pl.pallas_call(
        _ea_body,
        grid=(E // BE2,),
        in_specs=[
            pl.BlockSpec((BE2 // 4, 68), lambda i: (i, 0)),
            pl.BlockSpec((64, 8 * 128), lambda i: (0, 0)),
            pl.BlockSpec((4 * NG, 8 * 128), lambda i: (0, 0)),
        ],
        out_specs=[pl.BlockSpec((2, BE2 // 4, 128), lambda i: (0, i, 0))] * 4,
        out_shape=[jax.ShapeDtypeStruct((2, E4, 128), jnp.float32)] * 4,
    )(eaE, wbd, gtbd)


def _upd_body(h_ref, a0_ref, a1_ref, d0_ref, d1_ref, u1h_ref, w2u_ref,
              bvec_ref, u2_ref, b2u2_ref, lns_ref, lnb_ref, wx_ref,
              h_out, hx0_out, hx1_out):
    h = h_ref[...]
    w2u = w2u_ref[...]
    bv = bvec_ref[...]
    deg = d0_ref[0][:, :1] + d1_ref[0][:, :1]
    t = (h @ u1h_ref[...] + a0_ref[0] @ w2u[:32] + a1_ref[0] @ w2u[32:]
         + deg * bv[0:1] + bv[1:2])
    t = jnp.maximum(t, 0.0)
    u = t @ u2_ref[...] + b2u2_ref[...]
    u = _ln(u, lns_ref[...], lnb_ref[...])
    hn = h + u
    h_out[...] = hn
    hh = hn @ wx_ref[...]
    hx0_out[...] = hh[:, :32]
    hx1_out[...] = hh[:, 32:]


def _update(h, aggp, deg, u1h, w2u, bvec, u2, b2u2, lns, lnb, wx):
    return pl.pallas_call(
        _upd_body,
        grid=(NBLK,),
        in_specs=[
            pl.BlockSpec((BN, H), lambda i: (i, 0)),
            pl.BlockSpec((1, BN, 32), lambda i: (0, i, 0)),
            pl.BlockSpec((1, BN, 32), lambda i: (1, i, 0)),
            pl.BlockSpec((1, BN, 8), lambda i: (0, i, 0)),
            pl.BlockSpec((1, BN, 8), lambda i: (1, i, 0)),
            pl.BlockSpec((H, H), lambda i: (0, 0)),
            pl.BlockSpec((H, H), lambda i: (0, 0)),
            pl.BlockSpec((2, H), lambda i: (0, 0)),
            pl.BlockSpec((H, H), lambda i: (0, 0)),
            pl.BlockSpec((1, H), lambda i: (0, 0)),
            pl.BlockSpec((1, H), lambda i: (0, 0)),
            pl.BlockSpec((1, H), lambda i: (0, 0)),
            pl.BlockSpec((H, H), lambda i: (0, 0)),
        ],
        out_specs=[
            pl.BlockSpec((BN, H), lambda i: (i, 0)),
            pl.BlockSpec((BN, 32), lambda i: (i, 0)),
            pl.BlockSpec((BN, 32), lambda i: (i, 0)),
        ],
        out_shape=[
            jax.ShapeDtypeStruct((N, H), jnp.float32),
            jax.ShapeDtypeStruct((N, 32), jnp.float32),
            jax.ShapeDtypeStruct((N, 32), jnp.float32),
        ],
    )(h, aggp, aggp, deg, deg, u1h, w2u, bvec, u2, b2u2, lns, lnb, wx)


def _updpool_body(h_ref, a0_ref, a1_ref, d0_ref, d1_ref, u1h_ref, w2u_ref,
                  bvec_ref, u2_ref, b2u2_ref, lns_ref, lnb_ref, batch_ref,
                  s_out, m_out, c_out):
    i = pl.program_id(0)
    h = h_ref[...]
    w2u = w2u_ref[...]
    bv = bvec_ref[...]
    deg = d0_ref[0][:, :1] + d1_ref[0][:, :1]
    t = (h @ u1h_ref[...] + a0_ref[0] @ w2u[:32] + a1_ref[0] @ w2u[32:]
         + deg * bv[0:1] + bv[1:2])
    t = jnp.maximum(t, 0.0)
    u = t @ u2_ref[...] + b2u2_ref[...]
    u = _ln(u, lns_ref[...], lnb_ref[...])
    hn = h + u

    @pl.when(i == 0)
    def _():
        s_out[...] = jnp.zeros_like(s_out)
        m_out[...] = jnp.full_like(m_out, NEG)
        c_out[...] = jnp.zeros_like(c_out)

    rows = i * BN + lax.broadcasted_iota(jnp.int32, (BN, 1), 0)
    valid = rows < N
    hn = jnp.where(valid, hn, 0.0)
    bids = batch_ref[...]
    bi = lax.broadcasted_iota(jnp.int32, (BN, B), 1)
    msk = ((bids == bi) & valid).astype(jnp.float32)
    s_out[...] += lax.dot_general(msk, hn, (((0,), (0,)), ((), ())))
    c_out[...] += lax.dot_general(msk, jnp.ones((BN, H), jnp.float32),
                                  (((0,), (0,)), ((), ())))
    bmin = bids[0, 0]
    bmax = bids[BN - 1, 0]

    def body(b, _):
        sel = jnp.where((bids == b) & valid, hn, NEG)
        mx = jnp.max(sel, axis=0, keepdims=True)
        m_out[pl.ds(b, 1), :] = jnp.maximum(m_out[pl.ds(b, 1), :], mx)
        return 0

    lax.fori_loop(bmin, bmax + 1, body, 0)


def _update_pool(h, aggp, deg, u1h, w2u, bvec, u2, b2u2, lns, lnb, batch_p):
    return pl.pallas_call(
        _updpool_body,
        grid=(NBLK,),
        in_specs=[
            pl.BlockSpec((BN, H), lambda i: (i, 0)),
            pl.BlockSpec((1, BN, 32), lambda i: (0, i, 0)),
            pl.BlockSpec((1, BN, 32), lambda i: (1, i, 0)),
            pl.BlockSpec((1, BN, 8), lambda i: (0, i, 0)),
            pl.BlockSpec((1, BN, 8), lambda i: (1, i, 0)),
            pl.BlockSpec((H, H), lambda i: (0, 0)),
            pl.BlockSpec((H, H), lambda i: (0, 0)),
            pl.BlockSpec((2, H), lambda i: (0, 0)),
            pl.BlockSpec((H, H), lambda i: (0, 0)),
            pl.BlockSpec((1, H), lambda i: (0, 0)),
            pl.BlockSpec((1, H), lambda i: (0, 0)),
            pl.BlockSpec((1, H), lambda i: (0, 0)),
            pl.BlockSpec((BN, 1), lambda i: (i, 0)),
        ],
        out_specs=[
            pl.BlockSpec((B, H), lambda i: (0, 0)),
            pl.BlockSpec((B, H), lambda i: (0, 0)),
            pl.BlockSpec((B, H), lambda i: (0, 0)),
        ],
        out_shape=[
            jax.ShapeDtypeStruct((B, H), jnp.float32),
            jax.ShapeDtypeStruct((B, H), jnp.float32),
            jax.ShapeDtypeStruct((B, H), jnp.float32),
        ],
    )(h, aggp, aggp, deg, deg, u1h, w2u, bvec, u2, b2u2, lns, lnb, batch_p)


def _head_body(s_ref, m_ref, c_ref, gf_ref, gw_ref, gb_ref, gs_ref, glb_ref,
               w1_ref, b1_ref, w2_ref, b2_ref, wh_ref, bh_ref, out_ref):
    cnt = c_ref[...]
    h_sum = s_ref[...]
    mean = h_sum / jnp.maximum(cnt, 1.0)
    hmax = jnp.where(cnt > 0, m_ref[...], 0.0)
    g = jnp.maximum(gf_ref[...] @ gw_ref[...] + gb_ref[...], 0.0)
    g = _ln(g, gs_ref[...], glb_ref[...])
    w1 = w1_ref[...]
    c1 = (mean @ w1[:H] + hmax @ w1[H:2 * H] + h_sum @ w1[2 * H:3 * H]
          + g @ w1[3 * H:] + b1_ref[...])
    c1 = jnp.maximum(c1, 0.0)
    c2 = jnp.maximum(c1 @ w2_ref[...] + b2_ref[...], 0.0)
    out_ref[...] = c2 @ wh_ref[...] + bh_ref[...]


def _head(h_sum, h_max, cnt, gf, gw, gb, gs, glb, w1, b1, w2, b2, wh, bh):
    return pl.pallas_call(
        _head_body,
        out_shape=jax.ShapeDtypeStruct((B, 128), jnp.float32),
    )(h_sum, h_max, cnt, gf, gw, gb, gs, glb, w1, b1, w2, b2, wh, bh)


BS = 256
_SC_EC = E_PAD // 16
_SC_CH = _SC_EC // BS
_ROWS_SUB = N_PAD // 16
_MESH = plsc.VectorSubcoreMesh(core_axis_name="c", subcore_axis_name="s")


@functools.partial(
    pl.kernel,
    out_type=jax.ShapeDtypeStruct((2, N_PAD, 32), jnp.float32),
    mesh=_MESH,
    compiler_params=pltpu.CompilerParams(use_tc_tiling_on_sc=False),
    scratch_types=[
        pltpu.VMEM((2, 8, 128), jnp.int32),
        pltpu.VMEM((2, 8, 128), jnp.int32),
        pltpu.VMEM((2, BS, 32), jnp.float32),
        pltpu.VMEM((BS // 4, 128), jnp.float32),
        pltpu.VMEM_SHARED((N_PAD, 32), jnp.float32),
        pltpu.SemaphoreType.DMA,
        pltpu.SemaphoreType.DMA,
    ],
)
def _edge_sc(hx0_hbm, hx1_hbm, ea_hbm, src_hbm, dst_hbm, zer_hbm, out_hbm,
             sidx, didx, rows, eab, acc, gsem, ssem):
    c = lax.axis_index("c")
    s = lax.axis_index("s")

    pltpu.sync_copy(zer_hbm, rows.at[0])
    base = s * _ROWS_SUB
    for t in range(_ROWS_SUB // BS):
        pltpu.sync_copy(rows.at[0], acc.at[pl.ds(base + BS * t, BS)])
    rem = _ROWS_SUB % BS
    if rem:
        pltpu.sync_copy(rows.at[0, pl.ds(0, rem)],
                        acc.at[pl.ds(base + (_ROWS_SUB // BS) * BS, rem)])
    plsc.subcore_barrier()

    NQ = _SC_CH // 4

    def load_batch(k, pp):
        rowb = s * (_SC_EC // 128) + k * 8
        pltpu.sync_copy(src_hbm.at[pl.ds(rowb, 8)], sidx.at[pp])
        pltpu.sync_copy(dst_hbm.at[pl.ds(rowb, 8)], didx.at[pp])

    def issue_gather(pp, slot, rb):
        @pl.when(c == 0)
        def _():
            for j in range(2):
                pltpu.async_copy(hx0_hbm.at[sidx.at[pp, 2 * slot + j]],
                                 rows.at[rb, pl.ds(j * 128, 128)], gsem)

        @pl.when(c == 1)
        def _():
            for j in range(2):
                pltpu.async_copy(hx1_hbm.at[sidx.at[pp, 2 * slot + j]],
                                 rows.at[rb, pl.ds(j * 128, 128)], gsem)

    def load_ea(g):
        pltpu.sync_copy(
            ea_hbm.at[c, pl.ds((s * _SC_EC + g * BS) // 4, BS // 4)], eab)

    def do_chunk(g, i, pp, np_, has_next):
        b = i % 2
        pltpu.make_async_copy(zer_hbm, rows.at[b], gsem).wait()
        if has_next:
            issue_gather(pp if i < 3 else np_, (i + 1) % 4, 1 - b)

        @plsc.parallel_loop(0, BS, unroll=8)
        def _relu(r):
            for hh in range(2):
                p = r * 32 + hh * 16
                sl = pl.ds(hh * 16, 16)
                rows[b, r, sl] = jnp.maximum(
                    rows[b, r, sl] + eab[p // 128, pl.ds(p % 128, 16)], 0.0)

        scps = [pltpu.async_copy(rows.at[b, pl.ds(j * 128, 128)],
                                 acc.at[didx.at[pp, 2 * i + j]], ssem,
                                 add=True)
                for j in range(2)]
        if has_next:
            load_ea(g + 1)
        for cp in scps:
            cp.wait()

    def do_quad(k, pp, last=False):
        if not last:
            load_batch(k + 1, 1 - pp)
        for i in range(4):
            do_chunk(4 * k + i, i, pp, 1 - pp,
                     has_next=(not last) or i < 3)

    load_batch(0, 0)
    issue_gather(0, 0, 0)
    load_ea(0)

    def pair(kk, carry):
        do_quad(2 * kk, 0)
        do_quad(2 * kk + 1, 1)
        return carry

    lax.fori_loop(0, NQ // 2, pair, 0)
    do_quad(NQ - 1, 0, last=True)
    plsc.subcore_barrier()

    for t in range(_ROWS_SUB // BS):
        pltpu.sync_copy(acc.at[pl.ds(base + BS * t, BS)], rows.at[0])
        pltpu.sync_copy(rows.at[0], out_hbm.at[c, pl.ds(base + BS * t, BS)])
    if rem:
        pltpu.sync_copy(acc.at[pl.ds(base + (_ROWS_SUB // BS) * BS, rem)],
                        rows.at[0, pl.ds(0, rem)])
        pltpu.sync_copy(rows.at[0, pl.ds(0, rem)],
                        out_hbm.at[c, pl.ds(base + (_ROWS_SUB // BS) * BS, rem)])


_DG_EW = E_PAD // 32
_DG_CH = _DG_EW // BS


@functools.partial(
    pl.kernel,
    out_type=jax.ShapeDtypeStruct((2, N_PAD, 8), jnp.float32),
    mesh=_MESH,
    compiler_params=pltpu.CompilerParams(use_tc_tiling_on_sc=False),
    scratch_types=[
        pltpu.VMEM((2, 128), jnp.int32),
        pltpu.VMEM((128, 8), jnp.float32),
        pltpu.VMEM((BS, 8), jnp.float32),
        pltpu.VMEM_SHARED((N_PAD, 8), jnp.float32),
        pltpu.SemaphoreType.DMA,
    ],
)
def _deg_sc(dst_hbm, ones_hbm, zer_hbm, out_hbm, didx, ones, stage, acc, sem):
    c = lax.axis_index("c")
    s = lax.axis_index("s")
    wid = s * 2 + c
    pltpu.sync_copy(ones_hbm, ones)
    pltpu.sync_copy(zer_hbm, stage)
    base = s * _ROWS_SUB
    for t in range(_ROWS_SUB // BS):
        pltpu.sync_copy(stage, acc.at[pl.ds(base + BS * t, BS)])
    rem = _ROWS_SUB % BS
    if rem:
        pltpu.sync_copy(stage.at[pl.ds(0, rem)],
                        acc.at[pl.ds(base + (_ROWS_SUB // BS) * BS, rem)])
    plsc.subcore_barrier()

    def chunk(g, carry):
        ebase = wid * _DG_EW + g * BS
        for j in range(2):
            pltpu.sync_copy(dst_hbm.at[pl.ds(ebase + j * 128, 128)],
                            didx.at[j])
        for j in range(2):
            pltpu.sync_copy(ones, acc.at[didx.at[j]], add=True)
        return carry

    lax.fori_loop(0, _DG_CH, chunk, 0)
    plsc.subcore_barrier()

    for t in range(_ROWS_SUB // BS):
        pltpu.sync_copy(acc.at[pl.ds(base + BS * t, BS)], stage)
        pltpu.sync_copy(stage, out_hbm.at[c, pl.ds(base + BS * t, BS)])
    if rem:
        pltpu.sync_copy(acc.at[pl.ds(base + (_ROWS_SUB // BS) * BS, rem)],
                        stage.at[pl.ds(0, rem)])
        pltpu.sync_copy(stage.at[pl.ds(0, rem)],
                        out_hbm.at[c, pl.ds(base + (_ROWS_SUB // BS) * BS, rem)])


def kernel(x, edge_index, edge_attr, edge_gate_type, batch, global_features, params):
    p = params
    src = edge_index[0]
    dst = edge_index[1]
    src_p = jnp.pad(src, (0, E_PAD - E))
    dst_p = jnp.pad(dst, (0, E_PAD - E), constant_values=N)
    ea4 = edge_attr.reshape(E // 4, 4 * D)
    egt4f = edge_gate_type.astype(jnp.float32).reshape(E // 4, 4)
    eaE = jnp.concatenate([ea4, egt4f], axis=1)
    batch_p = jnp.pad(batch, (0, N_PAD - N), constant_values=B - 1)[:, None]

    w1 = p['msg_W1']
    w1x = w1[:, :H]
    wbd = jnp.zeros((4 * D, 8, 4, 32), jnp.float32)
    for l in range(L):
        for cc in range(2):
            for ss in range(4):
                wbd = wbd.at[D * ss:D * (ss + 1), 2 * l + cc, ss].set(
                    w1[l, 2 * H:, 32 * cc:32 * (cc + 1)])
    wbd = wbd.reshape(4 * D, 8 * 128)
    gtbd = jnp.zeros((4 * NG, 8, 4, 32), jnp.float32)
    for l in range(L):
        gtab = p['gate_embed'][l] @ w1[l, H:2 * H] + p['msg_b1'][l][None, :]
        for cc in range(2):
            for ss in range(4):
                gtbd = gtbd.at[NG * ss:NG * (ss + 1), 2 * l + cc, ss].set(
                    gtab[:, 32 * cc:32 * (cc + 1)])
    gtbd = gtbd.reshape(4 * NG, 8 * 128)
    u1 = p['upd_W1']
    u1h = u1[:, :H]
    w2u = jnp.einsum('lij,ljk->lik', p['msg_W2'], u1[:, H:])
    b2u = jnp.einsum('lj,ljk->lk', p['msg_b2'], u1[:, H:])
    bvec = jnp.stack([b2u, p['upd_b1']], axis=1)
    wh = jnp.zeros((H, 128), jnp.float32)
    wh = wh.at[:, :NT].set(p['thr_W']).at[:, NT:NT + 1].set(p['run_W'])
    bh = jnp.zeros((1, 128), jnp.float32)
    bh = bh.at[0, :NT].set(p['thr_b']).at[0, NT:NT + 1].set(p['run_b'])

    def r2(v):
        return v[None, :]

    h, hx0, hx1 = _embed(x, p['embed_W'], r2(p['embed_b']),
                         r2(p['embed_ln_s']), r2(p['embed_ln_b']), w1x[0])
    ea_layers = _ea_proj(eaE, wbd, gtbd)
    deg8 = _deg_sc(dst_p, jnp.ones((128, 8), jnp.float32),
                   jnp.zeros((BS, 8), jnp.float32))
    zer32 = jnp.zeros((BS, 32), jnp.float32)
    src2 = src_p.reshape(E_PAD // 128, 128)
    dst2 = dst_p.reshape(E_PAD // 128, 128)

    for l in range(L):
        aggp = _edge_sc(hx0, hx1, ea_layers[l], src2, dst2, zer32)
        args = (h, aggp, deg8, u1h[l], w2u[l], bvec[l], p['upd_W2'][l],
                r2(p['upd_b2'][l]), r2(p['mp_ln_s'][l]), r2(p['mp_ln_b'][l]))
        if l < L - 1:
            h, hx0, hx1 = _update(*args, w1x[l + 1])
        else:
            h_sum, h_max, cnt = _update_pool(*args, batch_p)

    out = _head(h_sum, h_max, cnt, global_features, p['glob_W'],
                r2(p['glob_b']), r2(p['glob_ln_s']), r2(p['glob_ln_b']),
                p['comb_W1'], jnp.zeros((1, 2 * H), jnp.float32) + p['comb_b1'],
                p['comb_W2'], r2(p['comb_b2']), wh, bh)
    return out[:, :NT], out[:, NT]
